# baseline (speedup 1.0000x reference)
import functools

import jax
import jax.numpy as jnp
from jax import lax
from jax.experimental import pallas as pl
from jax.experimental.pallas import tpu as pltpu
from jax.experimental.pallas import tpu_sc as plsc

NC = 2
NS = 16
NW = NC * NS

B = 16384
D = 32
BPW = B // NW
CHUNK = 128
NCHUNK = BPW // CHUNK

_mesh = plsc.VectorSubcoreMesh(
    core_axis_name="c", subcore_axis_name="s", num_cores=NC, num_subcores=NS
)


@functools.partial(
    pl.kernel,
    mesh=_mesh,
    out_type=jax.ShapeDtypeStruct((B,), jnp.float32),
    compiler_params=pltpu.CompilerParams(
        needs_layout_passes=False, use_tc_tiling_on_sc=False),
    scratch_types=[
        pltpu.VMEM((NCHUNK, CHUNK), jnp.int32),
        pltpu.VMEM((NCHUNK, CHUNK), jnp.int32),
        pltpu.VMEM((BPW, D), jnp.float32),
        pltpu.VMEM((BPW, D), jnp.float32),
        pltpu.VMEM((BPW,), jnp.float32),
        pltpu.SemaphoreType.DMA,
    ],
)
def _sc_dot_decoder(z_hbm, h_hbm, t_hbm, out_hbm,
                    idx_h, idx_t, rows_h, rows_t, out_v, sem):
    wid = lax.axis_index("s") * NC + lax.axis_index("c")
    base = wid * BPW

    pltpu.sync_copy(h_hbm.at[wid], idx_h)
    pltpu.sync_copy(t_hbm.at[wid], idx_t)

    copies = []
    for j in range(NCHUNK):
        copies.append(
            pltpu.async_copy(z_hbm.at[idx_h.at[j]],
                             rows_h.at[pl.ds(j * CHUNK, CHUNK)], sem))
        copies.append(
            pltpu.async_copy(z_hbm.at[idx_t.at[j]],
                             rows_t.at[pl.ds(j * CHUNK, CHUNK)], sem))
    for c in copies:
        c.wait()

    lane = jnp.arange(16, dtype=jnp.int32)

    def body(g, carry):
        row_idx = g * 16 + lane
        acc = None
        for d in range(D):
            col_idx = jnp.full((16,), d, dtype=jnp.int32)
            prod = (plsc.load_gather(rows_h, [row_idx, col_idx])
                    * plsc.load_gather(rows_t, [row_idx, col_idx]))
            acc = prod if acc is None else acc + prod
        out_v[pl.ds(g * 16, 16)] = acc
        return carry

    lax.fori_loop(0, BPW // 16, body, 0)

    pltpu.sync_copy(out_v, out_hbm.at[pl.ds(base, BPW)])


def kernel(z, triples):
    h = triples[:, 0].reshape(NW, NCHUNK, CHUNK)
    t = triples[:, 2].reshape(NW, NCHUNK, CHUNK)
    return _sc_dot_decoder(z, h, t)
